# Initial kernel scaffold; baseline (speedup 1.0000x reference)
import jax
import jax.numpy as jnp
from jax.experimental import pallas as pl


def kernel(x, edge_index, W1, b1, W2, b2, W3, b3):
    raise NotImplementedError("write your pallas kernel here")


# speedup vs baseline: 2.8738x; 2.8738x over previous
import dataclasses
import functools

import jax
import jax.numpy as jnp
from jax import lax
from jax.experimental import pallas as pl
from jax.experimental.pallas import tpu as pltpu
from jax.experimental.pallas import tpu_sc as plsc

N = 10000
NP = 10240
E = 320000
NTILE = 16
CH = 128
EPT = 20480
E_PAD = NTILE * EPT
NCHUNK = EPT // CH
RPT = NP // NTILE
BN = 1024

_MESH = plsc.VectorSubcoreMesh(core_axis_name="c", subcore_axis_name="s")

_CP = pltpu.CompilerParams()
if "needs_layout_passes" in pltpu.CompilerParams.__dataclass_fields__:
    _CP = dataclasses.replace(_CP, needs_layout_passes=False)


NPH = NP // 2


def _make_deg():
    def body(sd, out, ebuf, hist, totals, tmp, accv, shared, sem):
        lanes = lax.iota(jnp.int32, 16)
        ones = jnp.full((16,), 1.0, jnp.float32)
        c = lax.axis_index("c")
        s = lax.axis_index("s")
        pltpu.sync_copy(sd.at[pl.ds(c * E_PAD + s * EPT, EPT)], ebuf)

        for p in range(2):
            def zero(j, carry):
                for l in range(16):
                    hist[l, pl.ds(j * 16, 16)] = jnp.zeros((16,), jnp.float32)
                return carry

            lax.fori_loop(0, NPH // 16, zero, 0)

            off = p * NPH

            def count(i, carry):
                idx = ebuf[pl.ds(i * 16, 16)] - off
                mask = (idx >= 0) & (idx < NPH)
                idxc = jnp.clip(idx, 0, NPH - 1)
                plsc.addupdate_scatter(hist, [lanes, idxc], ones, mask=mask)
                return carry

            lax.fori_loop(0, EPT // 16, count, 0)

            def lane_reduce(j, carry):
                v = hist[0, pl.ds(j * 16, 16)]
                for l in range(1, 16):
                    v = v + hist[l, pl.ds(j * 16, 16)]
                totals[pl.ds(off + j * 16, 16)] = v
                return carry

            lax.fori_loop(0, NPH // 16, lane_reduce, 0)

        pltpu.sync_copy(totals, shared.at[pl.ds(s * NP, NP)])
        plsc.subcore_barrier()

        def tile_reduce(j, carry):
            accv[pl.ds(j * 16, 16)] = jnp.zeros((16,), jnp.float32)
            return carry

        lax.fori_loop(0, RPT // 16, tile_reduce, 0)
        for t in range(NTILE):
            pltpu.sync_copy(shared.at[pl.ds(t * NP + s * RPT, RPT)], tmp)

            def add_in(j, carry):
                accv[pl.ds(j * 16, 16)] = (accv[pl.ds(j * 16, 16)]
                                           + tmp[pl.ds(j * 16, 16)])
                return carry

            lax.fori_loop(0, RPT // 16, add_in, 0)
        pltpu.sync_copy(accv, out.at[pl.ds(c * NP + s * RPT, RPT)])

    return pl.kernel(
        body,
        out_type=jax.ShapeDtypeStruct((2 * NP,), jnp.float32),
        mesh=_MESH,
        scratch_types=[
            pltpu.VMEM((EPT,), jnp.int32),
            pltpu.VMEM((16, NPH), jnp.float32),
            pltpu.VMEM((NP,), jnp.float32),
            pltpu.VMEM((RPT,), jnp.float32),
            pltpu.VMEM((RPT,), jnp.float32),
            pltpu.VMEM_SHARED((NTILE * NP,), jnp.float32),
            pltpu.SemaphoreType.DMA,
        ],
        compiler_params=_CP,
    )


def _make_agg(table_rows, edge_split):
    fh = 128
    if edge_split:
        ept = (E_PAD // 2) // NTILE
    else:
        ept = E_PAD // NTILE
    nchunk = ept // CH

    def body(table, srcs, dst, out, idx_s, idx_d, rows, acc, sem):
        c = lax.axis_index("c")
        s = lax.axis_index("s")

        def zero(r, carry):
            for k in range(fh // 16):
                rows[r, pl.ds(k * 16, 16)] = jnp.zeros((16,), jnp.float32)
            return carry

        lax.fori_loop(0, CH, zero, 0)
        for j in range(RPT // CH):
            pltpu.sync_copy(rows, acc.at[pl.ds(s * RPT + j * CH, CH)])
        plsc.subcore_barrier()

        def step(i, carry):
            if edge_split:
                base = c * (E_PAD // 2) + s * ept + i * CH
                sbase = base
            else:
                base = s * ept + i * CH
                sbase = c * E_PAD + base
            pltpu.sync_copy(srcs.at[pl.ds(sbase, CH)], idx_s)
            pltpu.sync_copy(dst.at[pl.ds(base, CH)], idx_d)
            pltpu.async_copy(table.at[idx_s], rows, sem).wait()
            pltpu.sync_copy(rows, acc.at[idx_d], add=True)
            return carry

        lax.fori_loop(0, nchunk, step, 0)
        plsc.subcore_barrier()
        for j in range(RPT // CH):
            pltpu.sync_copy(acc.at[pl.ds(s * RPT + j * CH, CH)], rows)
            pltpu.sync_copy(rows, out.at[pl.ds(c * NP + s * RPT + j * CH, CH)])

    return pl.kernel(
        body,
        out_type=jax.ShapeDtypeStruct((2 * NP, fh), jnp.float32),
        mesh=_MESH,
        scratch_types=[
            pltpu.VMEM((CH,), jnp.int32),
            pltpu.VMEM((CH,), jnp.int32),
            pltpu.VMEM((CH, fh), jnp.float32),
            pltpu.VMEM_SHARED((NP, fh), jnp.float32),
            pltpu.SemaphoreType.DMA,
        ],
    )


def _rs(cnt):
    return lax.rsqrt(jnp.maximum(cnt, 1.0))


def _pre(x_pad, co):
    def body(x_ref, co_ref, o_ref):
        o_ref[...] = x_ref[...] * _rs(co_ref[...])

    nb = NP // BN
    return pl.pallas_call(
        body,
        grid=(nb,),
        in_specs=[
            pl.BlockSpec((BN, 128), lambda i: (i, 0)),
            pl.BlockSpec((BN, 1), lambda i: (i, 0)),
        ],
        out_specs=pl.BlockSpec((BN, 128), lambda i: (i, 0)),
        out_shape=jax.ShapeDtypeStruct((NP, 128), jnp.float32),
    )(x_pad, co)


def _mm_post(g, wa, wb, b, ci, co, kh, fo):
    foh = fo // 2

    def body(ga, gb, wa_r, wb_r, b_r, ci_r, co_r, o_ref):
        y = jnp.dot(ga[...], wa_r[...], preferred_element_type=jnp.float32)
        y += jnp.dot(gb[...], wb_r[...], preferred_element_type=jnp.float32)
        y = y * _rs(ci_r[...]) + b_r[...]
        y = jnp.maximum(y, 0.0)
        o_ref[...] = y * _rs(co_r[...])

    nb = NP // BN
    return pl.pallas_call(
        body,
        grid=(2, nb),
        in_specs=[
            pl.BlockSpec((BN, kh), lambda h, i: (i, 0)),
            pl.BlockSpec((BN, kh), lambda h, i: (nb + i, 0)),
            pl.BlockSpec((kh, foh), lambda h, i: (0, h)),
            pl.BlockSpec((kh, foh), lambda h, i: (0, h)),
            pl.BlockSpec((1, foh), lambda h, i: (0, h)),
            pl.BlockSpec((BN, 1), lambda h, i: (i, 0)),
            pl.BlockSpec((BN, 1), lambda h, i: (i, 0)),
        ],
        out_specs=pl.BlockSpec((BN, foh), lambda h, i: (h * nb + i, 0)),
        out_shape=jax.ShapeDtypeStruct((2 * NP, foh), jnp.float32),
    )(g, g, wa, wb, b, ci, co)


def _mm_plain(g, wa, wb, kh, fo):
    def body(ga, gb, wa_r, wb_r, o_ref):
        y = jnp.dot(ga[...], wa_r[...], preferred_element_type=jnp.float32)
        y += jnp.dot(gb[...], wb_r[...], preferred_element_type=jnp.float32)
        o_ref[...] = y

    nb = NP // BN
    return pl.pallas_call(
        body,
        grid=(nb,),
        in_specs=[
            pl.BlockSpec((BN, kh), lambda i: (i, 0)),
            pl.BlockSpec((BN, kh), lambda i: (nb + i, 0)),
            pl.BlockSpec((kh, fo), lambda i: (0, 0)),
            pl.BlockSpec((kh, fo), lambda i: (0, 0)),
        ],
        out_specs=pl.BlockSpec((BN, fo), lambda i: (i, 0)),
        out_shape=jax.ShapeDtypeStruct((NP, fo), jnp.float32),
    )(g, g, wa, wb)


def _post(q, ci, b3p):
    def body(ga, gb, ci_r, b_r, o_ref):
        y = ga[...] + gb[...]
        o_ref[...] = y * _rs(ci_r[...]) + b_r[...]

    nb = NP // BN
    return pl.pallas_call(
        body,
        grid=(nb,),
        in_specs=[
            pl.BlockSpec((BN, 128), lambda i: (i, 0)),
            pl.BlockSpec((BN, 128), lambda i: (nb + i, 0)),
            pl.BlockSpec((BN, 1), lambda i: (i, 0)),
            pl.BlockSpec((1, 128), lambda i: (0, 0)),
        ],
        out_specs=pl.BlockSpec((BN, 128), lambda i: (i, 0)),
        out_shape=jax.ShapeDtypeStruct((NP, 128), jnp.float32),
    )(q, q, ci, b3p)


_deg = _make_deg()
_agg_es = _make_agg(NP, True)
_agg_fs = _make_agg(2 * NP, False)


def kernel(x, edge_index, W1, b1, W2, b2, W3, b3):
    src = edge_index[0]
    dst = edge_index[1]
    pad = jnp.full((E_PAD - E,), N, jnp.int32)
    src_p = jnp.concatenate([src, pad])
    dst_p = jnp.concatenate([dst, pad])
    sd = jnp.concatenate([src_p, dst_p])
    srcs2 = jnp.concatenate([src_p, src_p + NP])
    x_pad = jnp.pad(x, ((0, NP - N), (0, 0)))
    w3p = jnp.pad(W3, ((0, 0), (0, 88)))
    b3p = jnp.pad(b3, (0, 88)).reshape(1, 128)

    cnt = _deg(sd)
    co = cnt[:NP].reshape(NP, 1)
    ci = cnt[NP:].reshape(NP, 1)

    t1 = _pre(x_pad, co)
    p1 = _agg_es(t1, src_p, dst_p)
    z1 = _mm_post(p1, W1, W1, b1.reshape(1, 256), ci, co, 128, 256)
    g2 = _agg_fs(z1, srcs2, dst_p)
    z2 = _mm_post(g2, W2[:128], W2[128:], b2.reshape(1, 256), ci, co, 128, 256)
    v = _mm_plain(z2, w3p[:128], w3p[128:], 128, 128)
    q3 = _agg_es(v, src_p, dst_p)
    out = _post(q3, ci, b3p)
    return out[:N, :40]


# speedup vs baseline: 3.9176x; 1.3632x over previous
import dataclasses
import functools

import jax
import jax.numpy as jnp
from jax import lax
from jax.experimental import pallas as pl
from jax.experimental.pallas import tpu as pltpu
from jax.experimental.pallas import tpu_sc as plsc

N = 10000
NP = 10240
E = 320000
NTILE = 16
CH = 128
EPT = 20480
E_PAD = NTILE * EPT
NCHUNK = EPT // CH
RPT = NP // NTILE
BN = 1024

_MESH = plsc.VectorSubcoreMesh(core_axis_name="c", subcore_axis_name="s")

_CP = pltpu.CompilerParams()
if "needs_layout_passes" in pltpu.CompilerParams.__dataclass_fields__:
    _CP = dataclasses.replace(_CP, needs_layout_passes=False)


NPH = NP // 2


def _make_deg():
    def body(sd, out, ebuf, hist, totals, tmp, accv, shared, sem):
        lanes = lax.iota(jnp.int32, 16)
        ones = jnp.full((16,), 1.0, jnp.float32)
        c = lax.axis_index("c")
        s = lax.axis_index("s")
        pltpu.sync_copy(sd.at[pl.ds(c * E_PAD + s * EPT, EPT)], ebuf)

        for p in range(2):
            def zero(j, carry):
                for l in range(16):
                    hist[l, pl.ds(j * 16, 16)] = jnp.zeros((16,), jnp.float32)
                return carry

            lax.fori_loop(0, NPH // 16, zero, 0)

            off = p * NPH

            def count(i, carry):
                idx = ebuf[pl.ds(i * 16, 16)] - off
                mask = (idx >= 0) & (idx < NPH)
                idxc = jnp.clip(idx, 0, NPH - 1)
                plsc.addupdate_scatter(hist, [lanes, idxc], ones, mask=mask)
                return carry

            lax.fori_loop(0, EPT // 16, count, 0)

            def lane_reduce(j, carry):
                v = hist[0, pl.ds(j * 16, 16)]
                for l in range(1, 16):
                    v = v + hist[l, pl.ds(j * 16, 16)]
                totals[pl.ds(off + j * 16, 16)] = v
                return carry

            lax.fori_loop(0, NPH // 16, lane_reduce, 0)

        pltpu.sync_copy(totals, shared.at[pl.ds(s * NP, NP)])
        plsc.subcore_barrier()

        def tile_reduce(j, carry):
            accv[pl.ds(j * 16, 16)] = jnp.zeros((16,), jnp.float32)
            return carry

        lax.fori_loop(0, RPT // 16, tile_reduce, 0)
        for t in range(NTILE):
            pltpu.sync_copy(shared.at[pl.ds(t * NP + s * RPT, RPT)], tmp)

            def add_in(j, carry):
                accv[pl.ds(j * 16, 16)] = (accv[pl.ds(j * 16, 16)]
                                           + tmp[pl.ds(j * 16, 16)])
                return carry

            lax.fori_loop(0, RPT // 16, add_in, 0)
        pltpu.sync_copy(accv, out.at[pl.ds(c * NP + s * RPT, RPT)])

    return pl.kernel(
        body,
        out_type=jax.ShapeDtypeStruct((2 * NP,), jnp.float32),
        mesh=_MESH,
        scratch_types=[
            pltpu.VMEM((EPT,), jnp.int32),
            pltpu.VMEM((16, NPH), jnp.float32),
            pltpu.VMEM((NP,), jnp.float32),
            pltpu.VMEM((RPT,), jnp.float32),
            pltpu.VMEM((RPT,), jnp.float32),
            pltpu.VMEM_SHARED((NTILE * NP,), jnp.float32),
            pltpu.SemaphoreType.DMA,
        ],
        compiler_params=_CP,
    )


NBUF = 2


def _make_agg(edge_split, ib):
    fh = 128
    if edge_split:
        ept = (E_PAD // 2) // NTILE
    else:
        ept = E_PAD // NTILE
    nchunk = ept // CH
    nblk = nchunk // ib
    assert nblk * ib == nchunk

    def body(table, srcs, dst, out, idx_s, idx_d, rows, acc, *sems):
        gsem = sems[:NBUF]
        ssem = sems[NBUF:2 * NBUF]
        isem = sems[2 * NBUF]
        c = lax.axis_index("c")
        s = lax.axis_index("s")

        def zero(r, carry):
            for k in range(fh // 16):
                rows[0, r, pl.ds(k * 16, 16)] = jnp.zeros((16,), jnp.float32)
            return carry

        lax.fori_loop(0, CH, zero, 0)
        for j in range(RPT // CH):
            pltpu.sync_copy(rows.at[0], acc.at[pl.ds(s * RPT + j * CH, CH)])
        plsc.subcore_barrier()

        if edge_split:
            cbase0 = c * (E_PAD // 2) + s * ept
            sbase0 = cbase0
        else:
            cbase0 = s * ept
            sbase0 = c * E_PAD + s * ept

        def block(bi, carry):
            sbase = pl.multiple_of(sbase0 + bi * ib * CH, ib * CH)
            cbase = pl.multiple_of(cbase0 + bi * ib * CH, ib * CH)
            idl = []
            for j in range(ib):
                idl.append(pltpu.async_copy(
                    srcs.at[pl.ds(sbase + j * CH, CH)], idx_s.at[j], isem))
                idl.append(pltpu.async_copy(
                    dst.at[pl.ds(cbase + j * CH, CH)], idx_d.at[j], isem))
            for d in idl:
                d.wait()
            gd = [None] * NBUF
            sd = [None] * NBUF
            for j in range(ib):
                b = j % NBUF
                if sd[b] is not None:
                    sd[b].wait()
                gd[b] = pltpu.async_copy(
                    table.at[idx_s.at[j]], rows.at[b], gsem[b])
                jj = j - (NBUF - 1)
                if jj >= 0:
                    bb = jj % NBUF
                    gd[bb].wait()
                    sd[bb] = pltpu.async_copy(
                        rows.at[bb], acc.at[idx_d.at[jj]], ssem[bb], add=True)
            for jj in range(ib - NBUF + 1, ib):
                bb = jj % NBUF
                gd[bb].wait()
                sd[bb] = pltpu.async_copy(
                    rows.at[bb], acc.at[idx_d.at[jj]], ssem[bb], add=True)
            for bb in range(NBUF):
                if sd[bb] is not None:
                    sd[bb].wait()
            return carry

        lax.fori_loop(0, nblk, block, 0)
        plsc.subcore_barrier()
        for j in range(RPT // CH):
            pltpu.sync_copy(acc.at[pl.ds(s * RPT + j * CH, CH)], rows.at[0])
            pltpu.sync_copy(rows.at[0],
                            out.at[pl.ds(c * NP + s * RPT + j * CH, CH)])

    return pl.kernel(
        body,
        out_type=jax.ShapeDtypeStruct((2 * NP, fh), jnp.float32),
        mesh=_MESH,
        scratch_types=[
            pltpu.VMEM((ib, CH), jnp.int32),
            pltpu.VMEM((ib, CH), jnp.int32),
            pltpu.VMEM((NBUF, CH, fh), jnp.float32),
            pltpu.VMEM_SHARED((NP, fh), jnp.float32),
        ] + [pltpu.SemaphoreType.DMA] * (2 * NBUF + 1),
    )


def _rs(cnt):
    return lax.rsqrt(jnp.maximum(cnt, 1.0))


def _pre(x_pad, co):
    def body(x_ref, co_ref, o_ref):
        o_ref[...] = x_ref[...] * _rs(co_ref[...])

    nb = NP // BN
    return pl.pallas_call(
        body,
        grid=(nb,),
        in_specs=[
            pl.BlockSpec((BN, 128), lambda i: (i, 0)),
            pl.BlockSpec((BN, 1), lambda i: (i, 0)),
        ],
        out_specs=pl.BlockSpec((BN, 128), lambda i: (i, 0)),
        out_shape=jax.ShapeDtypeStruct((NP, 128), jnp.float32),
    )(x_pad, co)


def _mm_post(g, wa, wb, b, ci, co, kh, fo):
    foh = fo // 2

    def body(ga, gb, wa_r, wb_r, b_r, ci_r, co_r, o_ref):
        y = jnp.dot(ga[...], wa_r[...], preferred_element_type=jnp.float32)
        y += jnp.dot(gb[...], wb_r[...], preferred_element_type=jnp.float32)
        y = y * _rs(ci_r[...]) + b_r[...]
        y = jnp.maximum(y, 0.0)
        o_ref[...] = y * _rs(co_r[...])

    nb = NP // BN
    return pl.pallas_call(
        body,
        grid=(2, nb),
        in_specs=[
            pl.BlockSpec((BN, kh), lambda h, i: (i, 0)),
            pl.BlockSpec((BN, kh), lambda h, i: (nb + i, 0)),
            pl.BlockSpec((kh, foh), lambda h, i: (0, h)),
            pl.BlockSpec((kh, foh), lambda h, i: (0, h)),
            pl.BlockSpec((1, foh), lambda h, i: (0, h)),
            pl.BlockSpec((BN, 1), lambda h, i: (i, 0)),
            pl.BlockSpec((BN, 1), lambda h, i: (i, 0)),
        ],
        out_specs=pl.BlockSpec((BN, foh), lambda h, i: (h * nb + i, 0)),
        out_shape=jax.ShapeDtypeStruct((2 * NP, foh), jnp.float32),
    )(g, g, wa, wb, b, ci, co)


def _mm_plain(g, wa, wb, kh, fo):
    def body(ga, gb, wa_r, wb_r, o_ref):
        y = jnp.dot(ga[...], wa_r[...], preferred_element_type=jnp.float32)
        y += jnp.dot(gb[...], wb_r[...], preferred_element_type=jnp.float32)
        o_ref[...] = y

    nb = NP // BN
    return pl.pallas_call(
        body,
        grid=(nb,),
        in_specs=[
            pl.BlockSpec((BN, kh), lambda i: (i, 0)),
            pl.BlockSpec((BN, kh), lambda i: (nb + i, 0)),
            pl.BlockSpec((kh, fo), lambda i: (0, 0)),
            pl.BlockSpec((kh, fo), lambda i: (0, 0)),
        ],
        out_specs=pl.BlockSpec((BN, fo), lambda i: (i, 0)),
        out_shape=jax.ShapeDtypeStruct((NP, fo), jnp.float32),
    )(g, g, wa, wb)


def _post(q, ci, b3p):
    def body(ga, gb, ci_r, b_r, o_ref):
        y = ga[...] + gb[...]
        o_ref[...] = y * _rs(ci_r[...]) + b_r[...]

    nb = NP // BN
    return pl.pallas_call(
        body,
        grid=(nb,),
        in_specs=[
            pl.BlockSpec((BN, 128), lambda i: (i, 0)),
            pl.BlockSpec((BN, 128), lambda i: (nb + i, 0)),
            pl.BlockSpec((BN, 1), lambda i: (i, 0)),
            pl.BlockSpec((1, 128), lambda i: (0, 0)),
        ],
        out_specs=pl.BlockSpec((BN, 128), lambda i: (i, 0)),
        out_shape=jax.ShapeDtypeStruct((NP, 128), jnp.float32),
    )(q, q, ci, b3p)


_deg = _make_deg()
_agg_es = _make_agg(True, 16)
_agg_fs = _make_agg(False, 16)


def kernel(x, edge_index, W1, b1, W2, b2, W3, b3):
    src = edge_index[0]
    dst = edge_index[1]
    pad = jnp.full((E_PAD - E,), N, jnp.int32)
    pad_d = N + (jnp.arange(E_PAD - E, dtype=jnp.int32) % (NP - N))
    src_p = jnp.concatenate([src, pad])
    dst_p = jnp.concatenate([dst, pad_d])
    sd = jnp.concatenate([src_p, dst_p])
    srcs2 = jnp.concatenate([src_p, src_p + NP])
    x_pad = jnp.pad(x, ((0, NP - N), (0, 0)))
    w3p = jnp.pad(W3, ((0, 0), (0, 88)))
    b3p = jnp.pad(b3, (0, 88)).reshape(1, 128)

    cnt = _deg(sd)
    co = cnt[:NP].reshape(NP, 1)
    ci = cnt[NP:].reshape(NP, 1)

    t1 = _pre(x_pad, co)
    p1 = _agg_es(t1, src_p, dst_p)
    z1 = _mm_post(p1, W1, W1, b1.reshape(1, 256), ci, co, 128, 256)
    g2 = _agg_fs(z1, srcs2, dst_p)
    z2 = _mm_post(g2, W2[:128], W2[128:], b2.reshape(1, 256), ci, co, 128, 256)
    v = _mm_plain(z2, w3p[:128], w3p[128:], 128, 128)
    q3 = _agg_es(v, src_p, dst_p)
    out = _post(q3, ci, b3p)
    return out[:N, :40]


# speedup vs baseline: 10.4947x; 2.6789x over previous
import dataclasses
import functools

import jax
import jax.numpy as jnp
from jax import lax
from jax.experimental import pallas as pl
from jax.experimental.pallas import tpu as pltpu
from jax.experimental.pallas import tpu_sc as plsc

N = 10000
NP = 10240
E = 320000
NTILE = 16
CH = 128
EPT = 20480
E_PAD = NTILE * EPT
NCHUNK = EPT // CH
RPT = NP // NTILE
BN = 1024

_MESH = plsc.VectorSubcoreMesh(core_axis_name="c", subcore_axis_name="s")

_CP = pltpu.CompilerParams()
if "needs_layout_passes" in pltpu.CompilerParams.__dataclass_fields__:
    _CP = dataclasses.replace(_CP, needs_layout_passes=False)


NPH = NP // 2


def _make_deg():
    def body(sd, out, ebuf, hist, totals, tmp, accv, shared, sem):
        lanes = lax.iota(jnp.int32, 16)
        ones = jnp.full((16,), 1.0, jnp.float32)
        c = lax.axis_index("c")
        s = lax.axis_index("s")
        pltpu.sync_copy(sd.at[pl.ds(c * E_PAD + s * EPT, EPT)], ebuf)

        for p in range(2):
            def zero(j, carry):
                for l in range(16):
                    hist[l, pl.ds(j * 16, 16)] = jnp.zeros((16,), jnp.float32)
                return carry

            lax.fori_loop(0, NPH // 16, zero, 0)

            off = p * NPH

            def count(i, carry):
                idx = ebuf[pl.ds(i * 16, 16)] - off
                mask = (idx >= 0) & (idx < NPH)
                idxc = jnp.clip(idx, 0, NPH - 1)
                plsc.addupdate_scatter(hist, [lanes, idxc], ones, mask=mask)
                return carry

            lax.fori_loop(0, EPT // 16, count, 0)

            def lane_reduce(j, carry):
                v = hist[0, pl.ds(j * 16, 16)]
                for l in range(1, 16):
                    v = v + hist[l, pl.ds(j * 16, 16)]
                totals[pl.ds(off + j * 16, 16)] = v
                return carry

            lax.fori_loop(0, NPH // 16, lane_reduce, 0)

        pltpu.sync_copy(totals, shared.at[pl.ds(s * NP, NP)])
        plsc.subcore_barrier()

        def tile_reduce(j, carry):
            accv[pl.ds(j * 16, 16)] = jnp.zeros((16,), jnp.float32)
            return carry

        lax.fori_loop(0, RPT // 16, tile_reduce, 0)
        for t in range(NTILE):
            pltpu.sync_copy(shared.at[pl.ds(t * NP + s * RPT, RPT)], tmp)

            def add_in(j, carry):
                accv[pl.ds(j * 16, 16)] = (accv[pl.ds(j * 16, 16)]
                                           + tmp[pl.ds(j * 16, 16)])
                return carry

            lax.fori_loop(0, RPT // 16, add_in, 0)
        pltpu.sync_copy(accv, out.at[pl.ds(c * NP + s * RPT, RPT)])

    return pl.kernel(
        body,
        out_type=jax.ShapeDtypeStruct((2 * NP,), jnp.float32),
        mesh=_MESH,
        scratch_types=[
            pltpu.VMEM((EPT,), jnp.int32),
            pltpu.VMEM((16, NPH), jnp.float32),
            pltpu.VMEM((NP,), jnp.float32),
            pltpu.VMEM((RPT,), jnp.float32),
            pltpu.VMEM((RPT,), jnp.float32),
            pltpu.VMEM_SHARED((NTILE * NP,), jnp.float32),
            pltpu.SemaphoreType.DMA,
        ],
        compiler_params=_CP,
    )


NBUF = 4


def _make_agg(edge_split, ib, ch):
    fh = 128
    if edge_split:
        ept = (E_PAD // 2) // NTILE
    else:
        ept = E_PAD // NTILE
    nchunk = ept // ch
    nblk = nchunk // ib
    assert nblk * ib == nchunk

    def body(table, srcs, dst, out, idx_s, idx_d, rows, acc, *sems):
        gsem = sems[:NBUF]
        ssem = sems[NBUF:2 * NBUF]
        isem = sems[2 * NBUF]
        c = lax.axis_index("c")
        s = lax.axis_index("s")

        def zero(r, carry):
            for k in range(fh // 16):
                rows[0, r, pl.ds(k * 16, 16)] = jnp.zeros((16,), jnp.float32)
            return carry

        lax.fori_loop(0, ch, zero, 0)
        for j in range(RPT // ch):
            pltpu.sync_copy(rows.at[0], acc.at[pl.ds(s * RPT + j * ch, ch)])
        plsc.subcore_barrier()

        if edge_split:
            cbase0 = c * (E_PAD // 2) + s * ept
            sbase0 = cbase0
        else:
            cbase0 = s * ept
            sbase0 = c * E_PAD + s * ept

        def block(bi, carry):
            sbase = pl.multiple_of(sbase0 + bi * ib * ch, ib * ch)
            cbase = pl.multiple_of(cbase0 + bi * ib * ch, ib * ch)
            idl = []
            for j in range(ib):
                idl.append(pltpu.async_copy(
                    srcs.at[pl.ds(sbase + j * ch, ch)], idx_s.at[j], isem))
                idl.append(pltpu.async_copy(
                    dst.at[pl.ds(cbase + j * ch, ch)], idx_d.at[j], isem))
            for d in idl:
                d.wait()
            gd = [None] * NBUF
            sd = [None] * NBUF
            for j in range(ib):
                b = j % NBUF
                if sd[b] is not None:
                    sd[b].wait()
                gd[b] = pltpu.async_copy(
                    table.at[idx_s.at[j]], rows.at[b], gsem[b])
                jj = j - (NBUF - 1)
                if jj >= 0:
                    bb = jj % NBUF
                    gd[bb].wait()
                    sd[bb] = pltpu.async_copy(
                        rows.at[bb], acc.at[idx_d.at[jj]], ssem[bb], add=True)
            for jj in range(ib - NBUF + 1, ib):
                bb = jj % NBUF
                gd[bb].wait()
                sd[bb] = pltpu.async_copy(
                    rows.at[bb], acc.at[idx_d.at[jj]], ssem[bb], add=True)
            for bb in range(NBUF):
                if sd[bb] is not None:
                    sd[bb].wait()
            return carry

        lax.fori_loop(0, nblk, block, 0)
        plsc.subcore_barrier()
        pltpu.sync_copy(acc.at[pl.ds(s * RPT, RPT)],
                        out.at[pl.ds(c * NP + s * RPT, RPT)])

    return pl.kernel(
        body,
        out_type=jax.ShapeDtypeStruct((2 * NP, fh), jnp.float32),
        mesh=_MESH,
        scratch_types=[
            pltpu.VMEM((ib, ch), jnp.int32),
            pltpu.VMEM((ib, ch), jnp.int32),
            pltpu.VMEM((NBUF, ch, fh), jnp.float32),
            pltpu.VMEM_SHARED((NP, fh), jnp.float32),
        ] + [pltpu.SemaphoreType.DMA] * (2 * NBUF + 1),
    )


def _rs(cnt):
    return lax.rsqrt(jnp.maximum(cnt, 1.0))


def _pre(x_pad, co):
    def body(x_ref, co_ref, o_ref):
        o_ref[...] = x_ref[...] * _rs(co_ref[...])

    nb = NP // BN
    return pl.pallas_call(
        body,
        grid=(nb,),
        in_specs=[
            pl.BlockSpec((BN, 128), lambda i: (i, 0)),
            pl.BlockSpec((BN, 1), lambda i: (i, 0)),
        ],
        out_specs=pl.BlockSpec((BN, 128), lambda i: (i, 0)),
        out_shape=jax.ShapeDtypeStruct((NP, 128), jnp.float32),
    )(x_pad, co)


def _mm_post(g, wa, wb, b, ci, co, kh, fo):
    foh = fo // 2

    def body(ga, gb, wa_r, wb_r, b_r, ci_r, co_r, o_ref):
        y = jnp.dot(ga[...], wa_r[...], preferred_element_type=jnp.float32)
        y += jnp.dot(gb[...], wb_r[...], preferred_element_type=jnp.float32)
        y = y * _rs(ci_r[...]) + b_r[...]
        y = jnp.maximum(y, 0.0)
        o_ref[...] = y * _rs(co_r[...])

    nb = NP // BN
    return pl.pallas_call(
        body,
        grid=(2, nb),
        in_specs=[
            pl.BlockSpec((BN, kh), lambda h, i: (i, 0)),
            pl.BlockSpec((BN, kh), lambda h, i: (nb + i, 0)),
            pl.BlockSpec((kh, foh), lambda h, i: (0, h)),
            pl.BlockSpec((kh, foh), lambda h, i: (0, h)),
            pl.BlockSpec((1, foh), lambda h, i: (0, h)),
            pl.BlockSpec((BN, 1), lambda h, i: (i, 0)),
            pl.BlockSpec((BN, 1), lambda h, i: (i, 0)),
        ],
        out_specs=pl.BlockSpec((BN, foh), lambda h, i: (h * nb + i, 0)),
        out_shape=jax.ShapeDtypeStruct((2 * NP, foh), jnp.float32),
    )(g, g, wa, wb, b, ci, co)


def _mm_plain(g, wa, wb, kh, fo):
    def body(ga, gb, wa_r, wb_r, o_ref):
        y = jnp.dot(ga[...], wa_r[...], preferred_element_type=jnp.float32)
        y += jnp.dot(gb[...], wb_r[...], preferred_element_type=jnp.float32)
        o_ref[...] = y

    nb = NP // BN
    return pl.pallas_call(
        body,
        grid=(nb,),
        in_specs=[
            pl.BlockSpec((BN, kh), lambda i: (i, 0)),
            pl.BlockSpec((BN, kh), lambda i: (nb + i, 0)),
            pl.BlockSpec((kh, fo), lambda i: (0, 0)),
            pl.BlockSpec((kh, fo), lambda i: (0, 0)),
        ],
        out_specs=pl.BlockSpec((BN, fo), lambda i: (i, 0)),
        out_shape=jax.ShapeDtypeStruct((NP, fo), jnp.float32),
    )(g, g, wa, wb)


def _post(q, ci, b3p):
    def body(ga, gb, ci_r, b_r, o_ref):
        y = ga[...] + gb[...]
        o_ref[...] = y * _rs(ci_r[...]) + b_r[...]

    nb = NP // BN
    return pl.pallas_call(
        body,
        grid=(nb,),
        in_specs=[
            pl.BlockSpec((BN, 128), lambda i: (i, 0)),
            pl.BlockSpec((BN, 128), lambda i: (nb + i, 0)),
            pl.BlockSpec((BN, 1), lambda i: (i, 0)),
            pl.BlockSpec((1, 128), lambda i: (0, 0)),
        ],
        out_specs=pl.BlockSpec((BN, 128), lambda i: (i, 0)),
        out_shape=jax.ShapeDtypeStruct((NP, 128), jnp.float32),
    )(q, q, ci, b3p)


_deg = _make_deg()
_agg_es = _make_agg(True, 32, 64)
_agg_fs = _make_agg(False, 32, 64)


def kernel(x, edge_index, W1, b1, W2, b2, W3, b3):
    src = edge_index[0]
    dst = edge_index[1]
    pad_c = N + (jnp.arange(E_PAD - E, dtype=jnp.int32) % (NP - N))
    src_p = jnp.concatenate([src, pad_c])
    dst_p = jnp.concatenate([dst, pad_c])
    sd = jnp.concatenate([src_p, dst_p])
    srcs2 = jnp.concatenate([src_p, src_p + NP])
    x_pad = jnp.pad(x, ((0, NP - N), (0, 0)))
    w3p = jnp.pad(W3, ((0, 0), (0, 88)))
    b3p = jnp.pad(b3, (0, 88)).reshape(1, 128)

    cnt = _deg(sd)
    co = cnt[:NP].reshape(NP, 1)
    ci = cnt[NP:].reshape(NP, 1)

    t1 = _pre(x_pad, co)
    p1 = _agg_es(t1, src_p, dst_p)
    z1 = _mm_post(p1, W1, W1, b1.reshape(1, 256), ci, co, 128, 256)
    g2 = _agg_fs(z1, srcs2, dst_p)
    z2 = _mm_post(g2, W2[:128], W2[128:], b2.reshape(1, 256), ci, co, 128, 256)
    v = _mm_plain(z2, w3p[:128], w3p[128:], 128, 128)
    q3 = _agg_es(v, src_p, dst_p)
    out = _post(q3, ci, b3p)
    return out[:N, :40]


# speedup vs baseline: 10.8877x; 1.0374x over previous
import dataclasses
import functools

import jax
import jax.numpy as jnp
from jax import lax
from jax.experimental import pallas as pl
from jax.experimental.pallas import tpu as pltpu
from jax.experimental.pallas import tpu_sc as plsc

N = 10000
NP = 10240
E = 320000
NTILE = 16
CH = 128
EPT = 20480
E_PAD = NTILE * EPT
NCHUNK = EPT // CH
RPT = NP // NTILE
BN = 1024

_MESH = plsc.VectorSubcoreMesh(core_axis_name="c", subcore_axis_name="s")

_CP = pltpu.CompilerParams()
if "needs_layout_passes" in pltpu.CompilerParams.__dataclass_fields__:
    _CP = dataclasses.replace(_CP, needs_layout_passes=False)


NPH = NP // 2


def _make_deg():
    def body(sd, out, ebuf, hist, totals, tmp, accv, shared, sem):
        lanes = lax.iota(jnp.int32, 16)
        ones = jnp.full((16,), 1.0, jnp.float32)
        c = lax.axis_index("c")
        s = lax.axis_index("s")
        pltpu.sync_copy(sd.at[pl.ds(c * E_PAD + s * EPT, EPT)], ebuf)

        for p in range(2):
            def zero(j, carry):
                for l in range(16):
                    hist[l, pl.ds(j * 16, 16)] = jnp.zeros((16,), jnp.float32)
                return carry

            lax.fori_loop(0, NPH // 16, zero, 0)

            off = p * NPH

            def count(i, carry):
                idx = ebuf[pl.ds(i * 16, 16)] - off
                mask = (idx >= 0) & (idx < NPH)
                idxc = jnp.clip(idx, 0, NPH - 1)
                plsc.addupdate_scatter(hist, [lanes, idxc], ones, mask=mask)
                return carry

            lax.fori_loop(0, EPT // 16, count, 0)

            def lane_reduce(j, carry):
                v = hist[0, pl.ds(j * 16, 16)]
                for l in range(1, 16):
                    v = v + hist[l, pl.ds(j * 16, 16)]
                totals[pl.ds(off + j * 16, 16)] = v
                return carry

            lax.fori_loop(0, NPH // 16, lane_reduce, 0)

        pltpu.sync_copy(totals, shared.at[pl.ds(s * NP, NP)])
        plsc.subcore_barrier()

        def tile_reduce(j, carry):
            accv[pl.ds(j * 16, 16)] = jnp.zeros((16,), jnp.float32)
            return carry

        lax.fori_loop(0, RPT // 16, tile_reduce, 0)
        for t in range(NTILE):
            pltpu.sync_copy(shared.at[pl.ds(t * NP + s * RPT, RPT)], tmp)

            def add_in(j, carry):
                accv[pl.ds(j * 16, 16)] = (accv[pl.ds(j * 16, 16)]
                                           + tmp[pl.ds(j * 16, 16)])
                return carry

            lax.fori_loop(0, RPT // 16, add_in, 0)
        pltpu.sync_copy(accv, out.at[pl.ds(c * NP + s * RPT, RPT)])

    return pl.kernel(
        body,
        out_type=jax.ShapeDtypeStruct((2 * NP,), jnp.float32),
        mesh=_MESH,
        scratch_types=[
            pltpu.VMEM((EPT,), jnp.int32),
            pltpu.VMEM((16, NPH), jnp.float32),
            pltpu.VMEM((NP,), jnp.float32),
            pltpu.VMEM((RPT,), jnp.float32),
            pltpu.VMEM((RPT,), jnp.float32),
            pltpu.VMEM_SHARED((NTILE * NP,), jnp.float32),
            pltpu.SemaphoreType.DMA,
        ],
        compiler_params=_CP,
    )


NBUF = 4


def _make_agg(edge_split, ib, ch):
    fh = 128
    if edge_split:
        ept = (E_PAD // 2) // NTILE
    else:
        ept = E_PAD // NTILE
    nchunk = ept // ch
    nblk = nchunk // ib
    assert nblk * ib == nchunk

    def body(table, srcs, dst, out, idx_s, idx_d, rows, acc, *sems):
        gsem = sems[:NBUF]
        ssem = sems[NBUF:2 * NBUF]
        isem = sems[2 * NBUF]
        c = lax.axis_index("c")
        s = lax.axis_index("s")

        def zero(r, carry):
            for k in range(fh // 16):
                rows[0, r, pl.ds(k * 16, 16)] = jnp.zeros((16,), jnp.float32)
            return carry

        lax.fori_loop(0, ch, zero, 0)
        for j in range(RPT // ch):
            pltpu.sync_copy(rows.at[0], acc.at[pl.ds(s * RPT + j * ch, ch)])
        plsc.subcore_barrier()

        if edge_split:
            cbase0 = c * (E_PAD // 2) + s * ept
            sbase0 = cbase0
        else:
            cbase0 = s * ept
            sbase0 = c * E_PAD + s * ept

        def block(bi, carry):
            sbase = pl.multiple_of(sbase0 + bi * ib * ch, ib * ch)
            cbase = pl.multiple_of(cbase0 + bi * ib * ch, ib * ch)
            idl = []
            for j in range(ib):
                idl.append(pltpu.async_copy(
                    srcs.at[pl.ds(sbase + j * ch, ch)], idx_s.at[j], isem))
                idl.append(pltpu.async_copy(
                    dst.at[pl.ds(cbase + j * ch, ch)], idx_d.at[j], isem))
            for d in idl:
                d.wait()
            gd = [None] * NBUF
            sd = [None] * NBUF
            for j in range(ib):
                b = j % NBUF
                if sd[b] is not None:
                    sd[b].wait()
                gd[b] = pltpu.async_copy(
                    table.at[idx_s.at[j]], rows.at[b], gsem[b])
                jj = j - (NBUF - 1)
                if jj >= 0:
                    bb = jj % NBUF
                    gd[bb].wait()
                    sd[bb] = pltpu.async_copy(
                        rows.at[bb], acc.at[idx_d.at[jj]], ssem[bb], add=True)
            for jj in range(ib - NBUF + 1, ib):
                bb = jj % NBUF
                gd[bb].wait()
                sd[bb] = pltpu.async_copy(
                    rows.at[bb], acc.at[idx_d.at[jj]], ssem[bb], add=True)
            for bb in range(NBUF):
                if sd[bb] is not None:
                    sd[bb].wait()
            return carry

        lax.fori_loop(0, nblk, block, 0)
        plsc.subcore_barrier()
        pltpu.sync_copy(acc.at[pl.ds(s * RPT, RPT)],
                        out.at[pl.ds(c * NP + s * RPT, RPT)])

    return pl.kernel(
        body,
        out_type=jax.ShapeDtypeStruct((2 * NP, fh), jnp.float32),
        mesh=_MESH,
        scratch_types=[
            pltpu.VMEM((ib, ch), jnp.int32),
            pltpu.VMEM((ib, ch), jnp.int32),
            pltpu.VMEM((NBUF, ch, fh), jnp.float32),
            pltpu.VMEM_SHARED((NP, fh), jnp.float32),
        ] + [pltpu.SemaphoreType.DMA] * (2 * NBUF + 1),
    )


def _rs(cnt):
    return lax.rsqrt(jnp.maximum(cnt, 1.0))


def _pre(x_pad, co):
    def body(x_ref, co_ref, o_ref):
        o_ref[...] = x_ref[...] * _rs(co_ref[...])

    nb = NP // BN
    return pl.pallas_call(
        body,
        grid=(nb,),
        in_specs=[
            pl.BlockSpec((BN, 128), lambda i: (i, 0)),
            pl.BlockSpec((BN, 1), lambda i: (i, 0)),
        ],
        out_specs=pl.BlockSpec((BN, 128), lambda i: (i, 0)),
        out_shape=jax.ShapeDtypeStruct((NP, 128), jnp.float32),
    )(x_pad, co)


def _mm_post(g, wa, wb, b, ci, co, kh, fo):
    foh = fo // 2

    def body(ga, gb, wa_r, wb_r, b_r, ci_r, co_r, o_ref):
        y = jnp.dot(ga[...], wa_r[...], preferred_element_type=jnp.float32)
        y += jnp.dot(gb[...], wb_r[...], preferred_element_type=jnp.float32)
        y = y * _rs(ci_r[...]) + b_r[...]
        y = jnp.maximum(y, 0.0)
        o_ref[...] = y * _rs(co_r[...])

    nb = NP // BN
    return pl.pallas_call(
        body,
        grid=(2, nb),
        in_specs=[
            pl.BlockSpec((BN, kh), lambda h, i: (i, 0)),
            pl.BlockSpec((BN, kh), lambda h, i: (nb + i, 0)),
            pl.BlockSpec((kh, foh), lambda h, i: (0, h)),
            pl.BlockSpec((kh, foh), lambda h, i: (0, h)),
            pl.BlockSpec((1, foh), lambda h, i: (0, h)),
            pl.BlockSpec((BN, 1), lambda h, i: (i, 0)),
            pl.BlockSpec((BN, 1), lambda h, i: (i, 0)),
        ],
        out_specs=pl.BlockSpec((BN, foh), lambda h, i: (h * nb + i, 0)),
        out_shape=jax.ShapeDtypeStruct((2 * NP, foh), jnp.float32),
    )(g, g, wa, wb, b, ci, co)


def _mm23(g, w2, b2v, w3, ci, co):
    def body(ga, gb, w2a, w2b, b_r, ci_r, co_r, w3_r, o_ref):
        y = jnp.dot(ga[...], w2a[...], preferred_element_type=jnp.float32)
        y += jnp.dot(gb[...], w2b[...], preferred_element_type=jnp.float32)
        y = y * _rs(ci_r[...]) + b_r[...]
        y = jnp.maximum(y, 0.0) * _rs(co_r[...])
        o_ref[...] = jnp.dot(y, w3_r[...], preferred_element_type=jnp.float32)

    nb = NP // BN
    return pl.pallas_call(
        body,
        grid=(nb,),
        in_specs=[
            pl.BlockSpec((BN, 128), lambda i: (i, 0)),
            pl.BlockSpec((BN, 128), lambda i: (nb + i, 0)),
            pl.BlockSpec((128, 256), lambda i: (0, 0)),
            pl.BlockSpec((128, 256), lambda i: (0, 0)),
            pl.BlockSpec((1, 256), lambda i: (0, 0)),
            pl.BlockSpec((BN, 1), lambda i: (i, 0)),
            pl.BlockSpec((BN, 1), lambda i: (i, 0)),
            pl.BlockSpec((256, 128), lambda i: (0, 0)),
        ],
        out_specs=pl.BlockSpec((BN, 128), lambda i: (i, 0)),
        out_shape=jax.ShapeDtypeStruct((NP, 128), jnp.float32),
    )(g, g, w2[:128], w2[128:], b2v, ci, co, w3)


def _post(q, ci, b3p):
    def body(ga, gb, ci_r, b_r, o_ref):
        y = ga[...] + gb[...]
        o_ref[...] = y * _rs(ci_r[...]) + b_r[...]

    nb = NP // BN
    return pl.pallas_call(
        body,
        grid=(nb,),
        in_specs=[
            pl.BlockSpec((BN, 128), lambda i: (i, 0)),
            pl.BlockSpec((BN, 128), lambda i: (nb + i, 0)),
            pl.BlockSpec((BN, 1), lambda i: (i, 0)),
            pl.BlockSpec((1, 128), lambda i: (0, 0)),
        ],
        out_specs=pl.BlockSpec((BN, 128), lambda i: (i, 0)),
        out_shape=jax.ShapeDtypeStruct((NP, 128), jnp.float32),
    )(q, q, ci, b3p)


_deg = _make_deg()
_agg_es = _make_agg(True, 32, 64)
_agg_fs = _make_agg(False, 32, 64)


def kernel(x, edge_index, W1, b1, W2, b2, W3, b3):
    src = edge_index[0]
    dst = edge_index[1]
    pad_c = N + (jnp.arange(E_PAD - E, dtype=jnp.int32) % (NP - N))
    src_p = jnp.concatenate([src, pad_c])
    dst_p = jnp.concatenate([dst, pad_c])
    sd = jnp.concatenate([src_p, dst_p])
    srcs2 = jnp.concatenate([src_p, src_p + NP])
    x_pad = jnp.pad(x, ((0, NP - N), (0, 0)))
    w3p = jnp.pad(W3, ((0, 0), (0, 88)))
    b3p = jnp.pad(b3, (0, 88)).reshape(1, 128)

    cnt = _deg(sd)
    co = cnt[:NP].reshape(NP, 1)
    ci = cnt[NP:].reshape(NP, 1)

    t1 = _pre(x_pad, co)
    p1 = _agg_es(t1, src_p, dst_p)
    z1 = _mm_post(p1, W1, W1, b1.reshape(1, 256), ci, co, 128, 256)
    g2 = _agg_fs(z1, srcs2, dst_p)
    v = _mm23(g2, W2, b2.reshape(1, 256), w3p, ci, co)
    q3 = _agg_es(v, src_p, dst_p)
    out = _post(q3, ci, b3p)
    return out[:N, :40]
